# baseline (speedup 1.0000x reference)
import functools
import numpy as np
import jax
import jax.numpy as jnp
from jax import lax
from jax.experimental import pallas as pl
from jax.experimental.pallas import tpu as pltpu
from jax.experimental.pallas import tpu_sc as plsc

_D = 256
_H = 2
_L = 2
_P = 4
_SPATIAL = ((64, 64), (32, 32))
_BS = 2
_NQ = _SPATIAL[0][0] * _SPATIAL[0][1]
_NV = sum(h * w for h, w in _SPATIAL)
_NCOMBO = _H * _L * _P
_K = _L * _P * 4
_NOUT = _BS * _NQ * _H

_BQ = 512

_combo_l = np.array([(c % 8) // 4 for c in range(_NCOMBO)])
_W_L = np.array([_SPATIAL[l][1] for l in _combo_l], np.float32)
_H_L = np.array([_SPATIAL[l][0] for l in _combo_l], np.float32)
_START_L = np.array([0 if l == 0 else _SPATIAL[0][0] * _SPATIAL[0][1]
                     for l in _combo_l], np.int32)


def _prep_body(q_ref, ql_ref, wsox_ref, wsoy_ref, bsox_ref, bsoy_ref,
               waw_ref, baw_ref, idx_ref, wt_ref):
    b = pl.program_id(0) // (_NQ // _BQ)
    q = q_ref[...]
    dn = (((1,), (1,)), ((), ()))
    X = lax.dot_general(q, wsox_ref[...], dn,
                        preferred_element_type=jnp.float32) + bsox_ref[...]
    Y = lax.dot_general(q, wsoy_ref[...], dn,
                        preferred_element_type=jnp.float32) + bsoy_ref[...]
    logits = lax.dot_general(q, waw_ref[...], dn,
                             preferred_element_type=jnp.float32) + baw_ref[...]
    l0 = logits[:, 0:8]
    l1 = logits[:, 8:16]
    e0 = jnp.exp(l0 - jnp.max(l0, axis=1, keepdims=True))
    e1 = jnp.exp(l1 - jnp.max(l1, axis=1, keepdims=True))
    a0 = e0 / jnp.sum(e0, axis=1, keepdims=True)
    a1 = e1 / jnp.sum(e1, axis=1, keepdims=True)
    aw = jnp.concatenate([a0, a1], axis=1)

    def combo_const(v0, v1, dtype):
        return jnp.concatenate(
            [jnp.full((1, 4), v0, dtype), jnp.full((1, 4), v1, dtype)] * 2,
            axis=1)

    wl = combo_const(_SPATIAL[0][1], _SPATIAL[1][1], jnp.float32)
    hl = combo_const(_SPATIAL[0][0], _SPATIAL[1][0], jnp.float32)
    base = (combo_const(0, _SPATIAL[0][0] * _SPATIAL[0][1], jnp.int32)
            + (b * _NV).astype(jnp.int32))
    wdim = wl.astype(jnp.int32)

    def expand(col, scale64, scale32):
        c64 = jnp.broadcast_to(ql_ref[:, col:col + 1] * scale64, (_BQ, 4))
        c32 = jnp.broadcast_to(ql_ref[:, col + 2:col + 3] * scale32, (_BQ, 4))
        return jnp.concatenate([c64, c32, c64, c32], axis=1)

    x = X + expand(0, float(_SPATIAL[0][1]), float(_SPATIAL[1][1])) - 0.5
    y = Y + expand(1, float(_SPATIAL[0][0]), float(_SPATIAL[1][0])) - 0.5

    x0 = jnp.floor(x)
    y0 = jnp.floor(y)
    fx1 = x - x0
    fx0 = 1.0 - fx1
    fy1 = y - y0
    fy0 = 1.0 - fy1

    idxs = []
    wts = []
    for dy, fy in ((0.0, fy0), (1.0, fy1)):
        yi = y0 + dy
        for dx, fx in ((0.0, fx0), (1.0, fx1)):
            xi = x0 + dx
            valid = ((xi >= 0.0) & (xi <= wl - 1.0)
                     & (yi >= 0.0) & (yi <= hl - 1.0))
            xc = jnp.clip(xi, 0.0, wl - 1.0).astype(jnp.int32)
            yc = jnp.clip(yi, 0.0, hl - 1.0).astype(jnp.int32)
            idxs.append(base + yc * wdim + xc)
            wts.append(aw * fx * fy * valid.astype(jnp.float32))
    idx_ref[...] = jnp.concatenate(idxs, axis=1)
    wt_ref[...] = jnp.concatenate(wts, axis=1)


def _proj_body(a_ref, w_ref, b_ref, o_ref):
    o_ref[...] = lax.dot_general(
        a_ref[...].astype(jnp.float32), w_ref[...], (((1,), (1,)), ((), ())),
        preferred_element_type=jnp.float32) + b_ref[...]


_NW = 32
_KC = _L * _P
_QROWS = _BS * _NQ
_PER_W2 = _QROWS // _NW
_CHQ = 2
_ROWS_CH = _CHQ * _H * 4 * _KC
_NCHUNK = _PER_W2 // _CHQ
_ENT_W = _PER_W2 * _H * 4 * _KC


def _sc_attend(value_flat, idx_flat, wts_flat):
    mesh = plsc.VectorSubcoreMesh(core_axis_name="c", subcore_axis_name="s")

    @functools.partial(
        pl.kernel,
        out_type=jax.ShapeDtypeStruct((_QROWS, _H * _D // 2), jnp.int32),
        mesh=mesh,
        scratch_types=[
            pltpu.VMEM((_ENT_W,), jnp.int32),
            pltpu.VMEM((_ENT_W,), jnp.float32),
            pltpu.VMEM((_ROWS_CH, _D // 2), jnp.int32),
            pltpu.VMEM((_ROWS_CH, _D // 2), jnp.int32),
            pltpu.VMEM((_CHQ, _H * _D // 2), jnp.int32),
            pltpu.SemaphoreType.DMA,
            pltpu.SemaphoreType.DMA,
        ],
        compiler_params=pltpu.CompilerParams(needs_layout_passes=False),
    )
    def k(value_hbm, idx_hbm, wts_hbm, out_hbm, idx_v, wts_v, bufa, bufb,
          out_v, sema, semb):
        wid = lax.axis_index("s") * 2 + lax.axis_index("c")
        out_base = wid * _PER_W2
        pltpu.sync_copy(idx_hbm.at[pl.ds(wid * _ENT_W, _ENT_W)], idx_v)
        pltpu.sync_copy(wts_hbm.at[pl.ds(wid * _ENT_W, _ENT_W)], wts_v)

        def start(g, buf, sem):
            pltpu.async_copy(
                value_hbm.at[idx_v.at[pl.ds(g * _ROWS_CH, _ROWS_CH)]],
                buf, sem)

        def drain(buf, sem):
            pltpu.make_async_copy(value_hbm.at[pl.ds(0, _ROWS_CH)], buf,
                                  sem).wait()

        def compute(g, buf):
            for q2 in range(_CHQ):
                for h in range(_H):
                    def mbody(jm, acc, q2=q2, h=h):
                        j = jm // _KC
                        m = jm % _KC
                        row = q2 * 64 + j * 16 + h * _KC + m
                        wvec = plsc.load_gather(
                            wts_v,
                            [jnp.full((16,), g * _ROWS_CH + row, jnp.int32)])
                        new = []
                        for k in range(8):
                            v32 = plsc.bitcast(buf[row, pl.ds(k * 16, 16)],
                                               jnp.bfloat16)
                            ve, vo = plsc.unpack(
                                v32, format=plsc.PackFormat.INTERLEAVED)
                            new.append(acc[2 * k] + wvec * ve)
                            new.append(acc[2 * k + 1] + wvec * vo)
                        return tuple(new)
                    acc = lax.fori_loop(
                        0, 4 * _KC, mbody,
                        tuple(jnp.zeros((16,), jnp.float32)
                              for _ in range(16)))
                    for k in range(8):
                        packed = plsc.pack(
                            acc[2 * k], acc[2 * k + 1],
                            format=plsc.PackFormat.INTERLEAVED)
                        out_v[q2, pl.ds(h * _D // 2 + k * 16, 16)] = (
                            plsc.bitcast(packed, jnp.int32))
            pltpu.sync_copy(out_v,
                            out_hbm.at[pl.ds(out_base + g * _CHQ, _CHQ)])

        start(0, bufa, sema)

        def pair(gp, carry):
            g0 = gp * 2
            start(g0 + 1, bufb, semb)
            drain(bufa, sema)
            compute(g0, bufa)

            @pl.when(g0 + 2 < _NCHUNK)
            def _():
                start(g0 + 2, bufa, sema)

            drain(bufb, semb)
            compute(g0 + 1, bufb)
            return carry

        lax.fori_loop(0, _NCHUNK // 2, pair, 0)

    return k(value_flat, idx_flat, wts_flat)


def _prep_call(query, query_location, W_so, b_so, W_aw, b_aw):
    q2 = query.reshape(_BS * _NQ, _D)
    ql2 = query_location.reshape(_BS * _NQ, _L * 2)
    grid = (_BS * _NQ // _BQ,)
    full = lambda i: (0, 0)
    row = lambda i: (i, 0)
    return pl.pallas_call(
        _prep_body,
        grid=grid,
        in_specs=[
            pl.BlockSpec((_BQ, _D), row),
            pl.BlockSpec((_BQ, _L * 2), row),
            pl.BlockSpec((_NCOMBO, _D), full),
            pl.BlockSpec((_NCOMBO, _D), full),
            pl.BlockSpec((1, _NCOMBO), full),
            pl.BlockSpec((1, _NCOMBO), full),
            pl.BlockSpec((_NCOMBO, _D), full),
            pl.BlockSpec((1, _NCOMBO), full),
        ],
        out_specs=[pl.BlockSpec((_BQ, _NCOMBO * 4), row)] * 2,
        out_shape=[
            jax.ShapeDtypeStruct((_BS * _NQ, _NCOMBO * 4), jnp.int32),
            jax.ShapeDtypeStruct((_BS * _NQ, _NCOMBO * 4), jnp.float32),
        ],
    )(q2, ql2,
      W_so[0::2], W_so[1::2],
      b_so[0::2].reshape(1, _NCOMBO), b_so[1::2].reshape(1, _NCOMBO),
      W_aw, b_aw.reshape(1, _NCOMBO))


def _proj_call(attn2, W_op, b_op):
    grid = (_BS * _NQ // _BQ,)
    return pl.pallas_call(
        _proj_body,
        grid=grid,
        in_specs=[
            pl.BlockSpec((_BQ, _H * _D), lambda i: (i, 0)),
            pl.BlockSpec((_D, _H * _D), lambda i: (0, 0)),
            pl.BlockSpec((1, _D), lambda i: (0, 0)),
        ],
        out_specs=pl.BlockSpec((_BQ, _D), lambda i: (i, 0)),
        out_shape=jax.ShapeDtypeStruct((_BS * _NQ, _D), jnp.float32),
    )(attn2, W_op, b_op.reshape(1, _D))


def kernel(query, value, query_location, spatial_shapes, level_start_index,
           W_so, b_so, W_aw, b_aw, W_op, b_op):
    idx, wt = _prep_call(query, query_location, W_so, b_so, W_aw, b_aw)
    value_bf = value.reshape(_BS * _NV, _D).astype(jnp.bfloat16)
    value_i32 = lax.bitcast_convert_type(
        value_bf.reshape(_BS * _NV, _D // 2, 2), jnp.int32)
    attn_i32 = _sc_attend(value_i32, idx.reshape(-1), wt.reshape(-1))
    attn2 = lax.bitcast_convert_type(attn_i32, jnp.bfloat16).reshape(
        _QROWS, _H * _D)
    out = _proj_call(attn2, W_op, b_op)
    return out.reshape(_BS, _NQ, _D)


# speedup vs baseline: 1.2209x; 1.2209x over previous
import functools
import numpy as np
import jax
import jax.numpy as jnp
from jax import lax
from jax.experimental import pallas as pl
from jax.experimental.pallas import tpu as pltpu
from jax.experimental.pallas import tpu_sc as plsc

_D = 256
_H = 2
_L = 2
_P = 4
_SPATIAL = ((64, 64), (32, 32))
_BS = 2
_NQ = _SPATIAL[0][0] * _SPATIAL[0][1]
_NV = sum(h * w for h, w in _SPATIAL)
_NCOMBO = _H * _L * _P
_K = _L * _P * 4
_NOUT = _BS * _NQ * _H

_BQ = 512

_combo_l = np.array([(c % 8) // 4 for c in range(_NCOMBO)])
_W_L = np.array([_SPATIAL[l][1] for l in _combo_l], np.float32)
_H_L = np.array([_SPATIAL[l][0] for l in _combo_l], np.float32)
_START_L = np.array([0 if l == 0 else _SPATIAL[0][0] * _SPATIAL[0][1]
                     for l in _combo_l], np.int32)


def _prep_body(q_ref, ql_ref, wsox_ref, wsoy_ref, bsox_ref, bsoy_ref,
               waw_ref, baw_ref, idx_ref, wt_ref):
    b = pl.program_id(0) // (_NQ // _BQ)
    q = q_ref[...]
    dn = (((1,), (1,)), ((), ()))
    X = lax.dot_general(q, wsox_ref[...], dn,
                        preferred_element_type=jnp.float32) + bsox_ref[...]
    Y = lax.dot_general(q, wsoy_ref[...], dn,
                        preferred_element_type=jnp.float32) + bsoy_ref[...]
    logits = lax.dot_general(q, waw_ref[...], dn,
                             preferred_element_type=jnp.float32) + baw_ref[...]
    l0 = logits[:, 0:8]
    l1 = logits[:, 8:16]
    e0 = jnp.exp(l0 - jnp.max(l0, axis=1, keepdims=True))
    e1 = jnp.exp(l1 - jnp.max(l1, axis=1, keepdims=True))
    a0 = e0 / jnp.sum(e0, axis=1, keepdims=True)
    a1 = e1 / jnp.sum(e1, axis=1, keepdims=True)
    aw = jnp.concatenate([a0, a1], axis=1)

    def combo_const(v0, v1, dtype):
        return jnp.concatenate(
            [jnp.full((1, 4), v0, dtype), jnp.full((1, 4), v1, dtype)] * 2,
            axis=1)

    wl = combo_const(_SPATIAL[0][1], _SPATIAL[1][1], jnp.float32)
    hl = combo_const(_SPATIAL[0][0], _SPATIAL[1][0], jnp.float32)
    base = (combo_const(0, _SPATIAL[0][0] * _SPATIAL[0][1], jnp.int32)
            + (b * _NV).astype(jnp.int32))
    wdim = wl.astype(jnp.int32)

    def expand(col, scale64, scale32):
        c64 = jnp.broadcast_to(ql_ref[:, col:col + 1] * scale64, (_BQ, 4))
        c32 = jnp.broadcast_to(ql_ref[:, col + 2:col + 3] * scale32, (_BQ, 4))
        return jnp.concatenate([c64, c32, c64, c32], axis=1)

    x = X + expand(0, float(_SPATIAL[0][1]), float(_SPATIAL[1][1])) - 0.5
    y = Y + expand(1, float(_SPATIAL[0][0]), float(_SPATIAL[1][0])) - 0.5

    x0 = jnp.floor(x)
    y0 = jnp.floor(y)
    fx1 = x - x0
    fx0 = 1.0 - fx1
    fy1 = y - y0
    fy0 = 1.0 - fy1

    idxs = []
    wts = []
    for dy, fy in ((0.0, fy0), (1.0, fy1)):
        yi = y0 + dy
        for dx, fx in ((0.0, fx0), (1.0, fx1)):
            xi = x0 + dx
            valid = ((xi >= 0.0) & (xi <= wl - 1.0)
                     & (yi >= 0.0) & (yi <= hl - 1.0))
            xc = jnp.clip(xi, 0.0, wl - 1.0).astype(jnp.int32)
            yc = jnp.clip(yi, 0.0, hl - 1.0).astype(jnp.int32)
            idxs.append(base + yc * wdim + xc)
            wts.append(aw * fx * fy * valid.astype(jnp.float32))
    idx_ref[...] = jnp.concatenate(idxs, axis=1)
    wt_ref[...] = jnp.concatenate(wts, axis=1)


def _proj_body(a_ref, w_ref, b_ref, o_ref):
    o_ref[...] = lax.dot_general(
        a_ref[...], w_ref[...], (((1,), (1,)), ((), ())),
        preferred_element_type=jnp.float32) + b_ref[...]


_NW = 32
_KC = _L * _P
_QROWS = _BS * _NQ
_PER_W2 = _QROWS // _NW
_CHQ = 2
_ROWS_CH = _CHQ * _H * 4 * _KC
_NCHUNK = _PER_W2 // _CHQ
_ENT_W = _PER_W2 * _H * 4 * _KC


def _sc_attend(value_flat, idx_flat, wts_flat):
    mesh = plsc.VectorSubcoreMesh(core_axis_name="c", subcore_axis_name="s")

    @functools.partial(
        pl.kernel,
        out_type=jax.ShapeDtypeStruct((_QROWS, _H * _D), jnp.float32),
        mesh=mesh,
        scratch_types=[
            pltpu.VMEM((_ENT_W,), jnp.int32),
            pltpu.VMEM((_ENT_W,), jnp.float32),
            pltpu.VMEM((_ROWS_CH, _D), jnp.float32),
            pltpu.VMEM((_ROWS_CH, _D), jnp.float32),
            pltpu.VMEM((_CHQ, _H * _D), jnp.float32),
            pltpu.SemaphoreType.DMA,
            pltpu.SemaphoreType.DMA,
        ],
        compiler_params=pltpu.CompilerParams(needs_layout_passes=False),
    )
    def k(value_hbm, idx_hbm, wts_hbm, out_hbm, idx_v, wts_v, bufa, bufb,
          out_v, sema, semb):
        wid = lax.axis_index("s") * 2 + lax.axis_index("c")
        out_base = wid * _PER_W2
        pltpu.sync_copy(idx_hbm.at[pl.ds(wid * _ENT_W, _ENT_W)], idx_v)
        pltpu.sync_copy(wts_hbm.at[pl.ds(wid * _ENT_W, _ENT_W)], wts_v)

        def start(g, buf, sem):
            pltpu.async_copy(
                value_hbm.at[idx_v.at[pl.ds(g * _ROWS_CH, _ROWS_CH)]],
                buf, sem)

        def drain(buf, sem):
            pltpu.make_async_copy(value_hbm.at[pl.ds(0, _ROWS_CH)], buf,
                                  sem).wait()

        def compute(g, buf):
            for q2 in range(_CHQ):
                for h in range(_H):
                    def mbody(jm, acc, q2=q2, h=h):
                        j = jm // _KC
                        m = jm % _KC
                        row = q2 * 64 + j * 16 + h * _KC + m
                        wvec = plsc.load_gather(
                            wts_v,
                            [jnp.full((16,), g * _ROWS_CH + row, jnp.int32)])
                        return tuple(
                            acc[c] + wvec * buf[row, pl.ds(c * 16, 16)]
                            for c in range(16))
                    acc = lax.fori_loop(
                        0, 4 * _KC, mbody,
                        tuple(jnp.zeros((16,), jnp.float32)
                              for _ in range(16)))
                    for c in range(16):
                        out_v[q2, pl.ds(h * _D + c * 16, 16)] = acc[c]
            pltpu.sync_copy(out_v,
                            out_hbm.at[pl.ds(out_base + g * _CHQ, _CHQ)])

        start(0, bufa, sema)

        def pair(gp, carry):
            g0 = gp * 2
            start(g0 + 1, bufb, semb)
            drain(bufa, sema)
            compute(g0, bufa)

            @pl.when(g0 + 2 < _NCHUNK)
            def _():
                start(g0 + 2, bufa, sema)

            drain(bufb, semb)
            compute(g0 + 1, bufb)
            return carry

        lax.fori_loop(0, _NCHUNK // 2, pair, 0)

    return k(value_flat, idx_flat, wts_flat)


def _prep_call(query, query_location, W_so, b_so, W_aw, b_aw):
    q2 = query.reshape(_BS * _NQ, _D)
    ql2 = query_location.reshape(_BS * _NQ, _L * 2)
    grid = (_BS * _NQ // _BQ,)
    full = lambda i: (0, 0)
    row = lambda i: (i, 0)
    return pl.pallas_call(
        _prep_body,
        grid=grid,
        in_specs=[
            pl.BlockSpec((_BQ, _D), row),
            pl.BlockSpec((_BQ, _L * 2), row),
            pl.BlockSpec((_NCOMBO, _D), full),
            pl.BlockSpec((_NCOMBO, _D), full),
            pl.BlockSpec((1, _NCOMBO), full),
            pl.BlockSpec((1, _NCOMBO), full),
            pl.BlockSpec((_NCOMBO, _D), full),
            pl.BlockSpec((1, _NCOMBO), full),
        ],
        out_specs=[pl.BlockSpec((_BQ, _NCOMBO * 4), row)] * 2,
        out_shape=[
            jax.ShapeDtypeStruct((_BS * _NQ, _NCOMBO * 4), jnp.int32),
            jax.ShapeDtypeStruct((_BS * _NQ, _NCOMBO * 4), jnp.float32),
        ],
    )(q2, ql2,
      W_so[0::2], W_so[1::2],
      b_so[0::2].reshape(1, _NCOMBO), b_so[1::2].reshape(1, _NCOMBO),
      W_aw, b_aw.reshape(1, _NCOMBO))


def _proj_call(attn2, W_op, b_op):
    grid = (_BS * _NQ // _BQ,)
    return pl.pallas_call(
        _proj_body,
        grid=grid,
        in_specs=[
            pl.BlockSpec((_BQ, _H * _D), lambda i: (i, 0)),
            pl.BlockSpec((_D, _H * _D), lambda i: (0, 0)),
            pl.BlockSpec((1, _D), lambda i: (0, 0)),
        ],
        out_specs=pl.BlockSpec((_BQ, _D), lambda i: (i, 0)),
        out_shape=jax.ShapeDtypeStruct((_BS * _NQ, _D), jnp.float32),
    )(attn2, W_op, b_op.reshape(1, _D))


def kernel(query, value, query_location, spatial_shapes, level_start_index,
           W_so, b_so, W_aw, b_aw, W_op, b_op):
    idx, wt = _prep_call(query, query_location, W_so, b_so, W_aw, b_aw)
    value_flat = value.reshape(_BS * _NV, _D)
    attn2 = _sc_attend(value_flat, idx.reshape(-1), wt.reshape(-1))
    out = _proj_call(attn2, W_op, b_op)
    return out.reshape(_BS, _NQ, _D)


# speedup vs baseline: 1.2832x; 1.0511x over previous
import functools
import numpy as np
import jax
import jax.numpy as jnp
from jax import lax
from jax.experimental import pallas as pl
from jax.experimental.pallas import tpu as pltpu
from jax.experimental.pallas import tpu_sc as plsc

_D = 256
_H = 2
_L = 2
_P = 4
_SPATIAL = ((64, 64), (32, 32))
_BS = 2
_NQ = _SPATIAL[0][0] * _SPATIAL[0][1]
_NV = sum(h * w for h, w in _SPATIAL)
_NCOMBO = _H * _L * _P
_K = _L * _P * 4
_NOUT = _BS * _NQ * _H

_BQ = 2048

_combo_l = np.array([(c % 8) // 4 for c in range(_NCOMBO)])
_W_L = np.array([_SPATIAL[l][1] for l in _combo_l], np.float32)
_H_L = np.array([_SPATIAL[l][0] for l in _combo_l], np.float32)
_START_L = np.array([0 if l == 0 else _SPATIAL[0][0] * _SPATIAL[0][1]
                     for l in _combo_l], np.int32)


def _prep_body(q_ref, ql_ref, wcat_ref, bcat_ref, idx_ref, wt_ref):
    b = pl.program_id(0) // (_NQ // _BQ)
    q = q_ref[...]
    dn = (((1,), (1,)), ((), ()))
    Z = lax.dot_general(q, wcat_ref[...], dn,
                        preferred_element_type=jnp.float32) + bcat_ref[...]
    X = Z[:, 0:_NCOMBO]
    Y = Z[:, _NCOMBO:2 * _NCOMBO]
    logits = Z[:, 2 * _NCOMBO:3 * _NCOMBO]
    l0 = logits[:, 0:8]
    l1 = logits[:, 8:16]
    e0 = jnp.exp(l0 - jnp.max(l0, axis=1, keepdims=True))
    e1 = jnp.exp(l1 - jnp.max(l1, axis=1, keepdims=True))
    a0 = e0 / jnp.sum(e0, axis=1, keepdims=True)
    a1 = e1 / jnp.sum(e1, axis=1, keepdims=True)
    aw = jnp.concatenate([a0, a1], axis=1)

    def combo_const(v0, v1, dtype):
        return jnp.concatenate(
            [jnp.full((1, 4), v0, dtype), jnp.full((1, 4), v1, dtype)] * 2,
            axis=1)

    wl = combo_const(_SPATIAL[0][1], _SPATIAL[1][1], jnp.float32)
    hl = combo_const(_SPATIAL[0][0], _SPATIAL[1][0], jnp.float32)
    base = (combo_const(0, _SPATIAL[0][0] * _SPATIAL[0][1], jnp.int32)
            + (b * _NV).astype(jnp.int32))
    wdim = wl.astype(jnp.int32)

    def expand(col, scale64, scale32):
        c64 = jnp.broadcast_to(ql_ref[:, col:col + 1] * scale64, (_BQ, 4))
        c32 = jnp.broadcast_to(ql_ref[:, col + 2:col + 3] * scale32, (_BQ, 4))
        return jnp.concatenate([c64, c32, c64, c32], axis=1)

    x = X + expand(0, float(_SPATIAL[0][1]), float(_SPATIAL[1][1])) - 0.5
    y = Y + expand(1, float(_SPATIAL[0][0]), float(_SPATIAL[1][0])) - 0.5

    x0 = jnp.floor(x)
    y0 = jnp.floor(y)
    fx1 = x - x0
    fx0 = 1.0 - fx1
    fy1 = y - y0
    fy0 = 1.0 - fy1

    idxs = []
    wts = []
    for dy, fy in ((0.0, fy0), (1.0, fy1)):
        yi = y0 + dy
        for dx, fx in ((0.0, fx0), (1.0, fx1)):
            xi = x0 + dx
            valid = ((xi >= 0.0) & (xi <= wl - 1.0)
                     & (yi >= 0.0) & (yi <= hl - 1.0))
            xc = jnp.clip(xi, 0.0, wl - 1.0).astype(jnp.int32)
            yc = jnp.clip(yi, 0.0, hl - 1.0).astype(jnp.int32)
            idxs.append(base + yc * wdim + xc)
            wts.append(aw * fx * fy * valid.astype(jnp.float32))
    idx_ref[...] = jnp.concatenate(idxs, axis=1)
    wt_ref[...] = jnp.concatenate(wts, axis=1)


def _proj_body(a_ref, w_ref, b_ref, o_ref):
    o_ref[...] = lax.dot_general(
        a_ref[...], w_ref[...], (((1,), (1,)), ((), ())),
        preferred_element_type=jnp.float32) + b_ref[...]


_NW = 32
_KC = _L * _P
_QROWS = _BS * _NQ
_PER_W2 = _QROWS // _NW
_CHQ = 2
_ROWS_CH = _CHQ * _H * 4 * _KC
_NCHUNK = _PER_W2 // _CHQ
_ENT_W = _PER_W2 * _H * 4 * _KC


def _sc_attend(value_flat, idx_flat, wts_flat):
    mesh = plsc.VectorSubcoreMesh(core_axis_name="c", subcore_axis_name="s")

    @functools.partial(
        pl.kernel,
        out_type=jax.ShapeDtypeStruct((_QROWS, _H * _D), jnp.float32),
        mesh=mesh,
        scratch_types=[
            pltpu.VMEM((_ENT_W,), jnp.int32),
            pltpu.VMEM((_ENT_W,), jnp.float32),
            pltpu.VMEM((_ROWS_CH, _D), jnp.float32),
            pltpu.VMEM((_ROWS_CH, _D), jnp.float32),
            pltpu.VMEM((_CHQ, _H * _D), jnp.float32),
            pltpu.SemaphoreType.DMA,
            pltpu.SemaphoreType.DMA,
        ],
        compiler_params=pltpu.CompilerParams(needs_layout_passes=False),
    )
    def k(value_hbm, idx_hbm, wts_hbm, out_hbm, idx_v, wts_v, bufa, bufb,
          out_v, sema, semb):
        wid = lax.axis_index("s") * 2 + lax.axis_index("c")
        out_base = wid * _PER_W2
        pltpu.sync_copy(idx_hbm.at[pl.ds(wid * _ENT_W, _ENT_W)], idx_v)
        pltpu.sync_copy(wts_hbm.at[pl.ds(wid * _ENT_W, _ENT_W)], wts_v)

        def start(g, buf, sem):
            pltpu.async_copy(
                value_hbm.at[idx_v.at[pl.ds(g * _ROWS_CH, _ROWS_CH)]],
                buf, sem)

        def drain(buf, sem):
            pltpu.make_async_copy(value_hbm.at[pl.ds(0, _ROWS_CH)], buf,
                                  sem).wait()

        def compute(g, buf):
            for q2 in range(_CHQ):
                for h in range(_H):
                    def mbody(jm, acc, q2=q2, h=h):
                        j = jm // _KC
                        m = jm % _KC
                        row = q2 * 64 + j * 16 + h * _KC + m
                        wvec = plsc.load_gather(
                            wts_v,
                            [jnp.full((16,), g * _ROWS_CH + row, jnp.int32)])
                        return tuple(
                            acc[c] + wvec * buf[row, pl.ds(c * 16, 16)]
                            for c in range(16))
                    acc = lax.fori_loop(
                        0, 4 * _KC, mbody,
                        tuple(jnp.zeros((16,), jnp.float32)
                              for _ in range(16)))
                    for c in range(16):
                        out_v[q2, pl.ds(h * _D + c * 16, 16)] = acc[c]
            pltpu.sync_copy(out_v,
                            out_hbm.at[pl.ds(out_base + g * _CHQ, _CHQ)])

        start(0, bufa, sema)

        def pair(gp, carry):
            g0 = gp * 2
            start(g0 + 1, bufb, semb)
            drain(bufa, sema)
            compute(g0, bufa)

            @pl.when(g0 + 2 < _NCHUNK)
            def _():
                start(g0 + 2, bufa, sema)

            drain(bufb, semb)
            compute(g0 + 1, bufb)
            return carry

        lax.fori_loop(0, _NCHUNK // 2, pair, 0)

    return k(value_flat, idx_flat, wts_flat)


def _prep_call(query, query_location, W_so, b_so, W_aw, b_aw):
    q2 = query.reshape(_BS * _NQ, _D)
    ql2 = query_location.reshape(_BS * _NQ, _L * 2)
    grid = (_BS * _NQ // _BQ,)
    full = lambda i: (0, 0)
    row = lambda i: (i, 0)
    return pl.pallas_call(
        _prep_body,
        grid=grid,
        in_specs=[
            pl.BlockSpec((_BQ, _D), row),
            pl.BlockSpec((_BQ, _L * 2), row),
            pl.BlockSpec((3 * _NCOMBO, _D), full),
            pl.BlockSpec((1, 3 * _NCOMBO), full),
        ],
        out_specs=[pl.BlockSpec((_BQ, _NCOMBO * 4), row)] * 2,
        out_shape=[
            jax.ShapeDtypeStruct((_BS * _NQ, _NCOMBO * 4), jnp.int32),
            jax.ShapeDtypeStruct((_BS * _NQ, _NCOMBO * 4), jnp.float32),
        ],
    )(q2, ql2,
      jnp.concatenate([W_so[0::2], W_so[1::2], W_aw], axis=0),
      jnp.concatenate([b_so[0::2], b_so[1::2], b_aw]).reshape(1, 3 * _NCOMBO))


def _proj_call(attn2, W_op, b_op):
    grid = (_BS * _NQ // _BQ,)
    return pl.pallas_call(
        _proj_body,
        grid=grid,
        in_specs=[
            pl.BlockSpec((_BQ, _H * _D), lambda i: (i, 0)),
            pl.BlockSpec((_D, _H * _D), lambda i: (0, 0)),
            pl.BlockSpec((1, _D), lambda i: (0, 0)),
        ],
        out_specs=pl.BlockSpec((_BQ, _D), lambda i: (i, 0)),
        out_shape=jax.ShapeDtypeStruct((_BS * _NQ, _D), jnp.float32),
    )(attn2, W_op, b_op.reshape(1, _D))


def kernel(query, value, query_location, spatial_shapes, level_start_index,
           W_so, b_so, W_aw, b_aw, W_op, b_op):
    idx, wt = _prep_call(query, query_location, W_so, b_so, W_aw, b_aw)
    value_flat = value.reshape(_BS * _NV, _D)
    attn2 = _sc_attend(value_flat, idx.reshape(-1), wt.reshape(-1))
    out = _proj_call(attn2, W_op, b_op)
    return out.reshape(_BS, _NQ, _D)


# speedup vs baseline: 1.2907x; 1.0058x over previous
import functools
import numpy as np
import jax
import jax.numpy as jnp
from jax import lax
from jax.experimental import pallas as pl
from jax.experimental.pallas import tpu as pltpu
from jax.experimental.pallas import tpu_sc as plsc

_D = 256
_H = 2
_L = 2
_P = 4
_SPATIAL = ((64, 64), (32, 32))
_BS = 2
_NQ = _SPATIAL[0][0] * _SPATIAL[0][1]
_NV = sum(h * w for h, w in _SPATIAL)
_NCOMBO = _H * _L * _P
_K = _L * _P * 4
_NOUT = _BS * _NQ * _H

_BQ = 2048

_combo_l = np.array([(c % 8) // 4 for c in range(_NCOMBO)])
_W_L = np.array([_SPATIAL[l][1] for l in _combo_l], np.float32)
_H_L = np.array([_SPATIAL[l][0] for l in _combo_l], np.float32)
_START_L = np.array([0 if l == 0 else _SPATIAL[0][0] * _SPATIAL[0][1]
                     for l in _combo_l], np.int32)


def _prep_body(q_ref, ql_ref, wcat_ref, bcat_ref, idx_ref, wt_ref):
    b = pl.program_id(0) // (_NQ // _BQ)
    q = q_ref[...]
    dn = (((1,), (1,)), ((), ()))
    Z = lax.dot_general(q, wcat_ref[...], dn,
                        preferred_element_type=jnp.float32) + bcat_ref[...]
    X = Z[:, 0:_NCOMBO]
    Y = Z[:, _NCOMBO:2 * _NCOMBO]
    logits = Z[:, 2 * _NCOMBO:3 * _NCOMBO]
    l0 = logits[:, 0:8]
    l1 = logits[:, 8:16]
    e0 = jnp.exp(l0 - jnp.max(l0, axis=1, keepdims=True))
    e1 = jnp.exp(l1 - jnp.max(l1, axis=1, keepdims=True))
    a0 = e0 / jnp.sum(e0, axis=1, keepdims=True)
    a1 = e1 / jnp.sum(e1, axis=1, keepdims=True)
    aw = jnp.concatenate([a0, a1], axis=1)

    def combo_const(v0, v1, dtype):
        return jnp.concatenate(
            [jnp.full((1, 4), v0, dtype), jnp.full((1, 4), v1, dtype)] * 2,
            axis=1)

    wl = combo_const(_SPATIAL[0][1], _SPATIAL[1][1], jnp.float32)
    hl = combo_const(_SPATIAL[0][0], _SPATIAL[1][0], jnp.float32)
    base = (combo_const(0, _SPATIAL[0][0] * _SPATIAL[0][1], jnp.int32)
            + (b * _NV).astype(jnp.int32))
    wdim = wl.astype(jnp.int32)

    def expand(col, scale64, scale32):
        c64 = jnp.broadcast_to(ql_ref[:, col:col + 1] * scale64, (_BQ, 4))
        c32 = jnp.broadcast_to(ql_ref[:, col + 2:col + 3] * scale32, (_BQ, 4))
        return jnp.concatenate([c64, c32, c64, c32], axis=1)

    x = X + expand(0, float(_SPATIAL[0][1]), float(_SPATIAL[1][1])) - 0.5
    y = Y + expand(1, float(_SPATIAL[0][0]), float(_SPATIAL[1][0])) - 0.5

    x0 = jnp.floor(x)
    y0 = jnp.floor(y)
    fx1 = x - x0
    fx0 = 1.0 - fx1
    fy1 = y - y0
    fy0 = 1.0 - fy1

    idxs = []
    wts = []
    for dy, fy in ((0.0, fy0), (1.0, fy1)):
        yi = y0 + dy
        for dx, fx in ((0.0, fx0), (1.0, fx1)):
            xi = x0 + dx
            valid = ((xi >= 0.0) & (xi <= wl - 1.0)
                     & (yi >= 0.0) & (yi <= hl - 1.0))
            xc = jnp.clip(xi, 0.0, wl - 1.0).astype(jnp.int32)
            yc = jnp.clip(yi, 0.0, hl - 1.0).astype(jnp.int32)
            idxs.append(base + yc * wdim + xc)
            wts.append(aw * fx * fy * valid.astype(jnp.float32))
    idx_ref[...] = jnp.concatenate(idxs, axis=1)
    wt_ref[...] = jnp.concatenate(wts, axis=1)


def _proj_body(a_ref, w_ref, b_ref, o_ref):
    o_ref[...] = lax.dot_general(
        a_ref[...], w_ref[...], (((1,), (1,)), ((), ())),
        preferred_element_type=jnp.float32) + b_ref[...]


_NW = 32
_KC = _L * _P
_QROWS = _BS * _NQ
_PER_W2 = _QROWS // _NW
_CHQ = 2
_ROWS_CH = _CHQ * _H * 4 * _KC
_NCHUNK = _PER_W2 // _CHQ
_ENT_W = _PER_W2 * _H * 4 * _KC


def _sc_attend(value_flat, idx_flat, wts_flat):
    mesh = plsc.VectorSubcoreMesh(core_axis_name="c", subcore_axis_name="s")

    @functools.partial(
        pl.kernel,
        out_type=jax.ShapeDtypeStruct((_QROWS, _H * _D), jnp.float32),
        mesh=mesh,
        scratch_types=[
            pltpu.VMEM((_ENT_W,), jnp.int32),
            pltpu.VMEM((_ENT_W,), jnp.float32),
            pltpu.VMEM((_ROWS_CH, _D), jnp.float32),
            pltpu.VMEM((_ROWS_CH, _D), jnp.float32),
            pltpu.VMEM((_CHQ, _H * _D), jnp.float32),
            pltpu.VMEM((_CHQ, _H * _D), jnp.float32),
            pltpu.SemaphoreType.DMA,
            pltpu.SemaphoreType.DMA,
            pltpu.SemaphoreType.DMA,
            pltpu.SemaphoreType.DMA,
        ],
        compiler_params=pltpu.CompilerParams(needs_layout_passes=False),
    )
    def k(value_hbm, idx_hbm, wts_hbm, out_hbm, idx_v, wts_v, bufa, bufb,
          out_va, out_vb, sema, semb, semoa, semob):
        wid = lax.axis_index("s") * 2 + lax.axis_index("c")
        out_base = wid * _PER_W2
        pltpu.sync_copy(idx_hbm.at[pl.ds(wid * _ENT_W, _ENT_W)], idx_v)
        pltpu.sync_copy(wts_hbm.at[pl.ds(wid * _ENT_W, _ENT_W)], wts_v)

        def start(g, buf, sem):
            pltpu.async_copy(
                value_hbm.at[idx_v.at[pl.ds(g * _ROWS_CH, _ROWS_CH)]],
                buf, sem)

        def drain(buf, sem):
            pltpu.make_async_copy(value_hbm.at[pl.ds(0, _ROWS_CH)], buf,
                                  sem).wait()

        def compute(g, buf, ov, semo):
            for q2 in range(_CHQ):
                for h in range(_H):
                    def mbody(jm, acc, q2=q2, h=h):
                        j = jm // _KC
                        m = jm % _KC
                        row = q2 * 64 + j * 16 + h * _KC + m
                        wvec = plsc.load_gather(
                            wts_v,
                            [jnp.full((16,), g * _ROWS_CH + row, jnp.int32)])
                        return tuple(
                            acc[c] + wvec * buf[row, pl.ds(c * 16, 16)]
                            for c in range(16))
                    acc = lax.fori_loop(
                        0, 4 * _KC, mbody,
                        tuple(jnp.zeros((16,), jnp.float32)
                              for _ in range(16)))
                    for c in range(16):
                        ov[q2, pl.ds(h * _D + c * 16, 16)] = acc[c]
            pltpu.async_copy(ov, out_hbm.at[pl.ds(out_base + g * _CHQ, _CHQ)],
                             semo)

        def wait_out(ov, semo):
            pltpu.make_async_copy(ov, out_hbm.at[pl.ds(0, _CHQ)], semo).wait()

        start(0, bufa, sema)

        def pair(gp, carry):
            g0 = gp * 2
            start(g0 + 1, bufb, semb)
            drain(bufa, sema)

            @pl.when(g0 >= 2)
            def _():
                wait_out(out_va, semoa)

            compute(g0, bufa, out_va, semoa)

            @pl.when(g0 + 2 < _NCHUNK)
            def _():
                start(g0 + 2, bufa, sema)

            drain(bufb, semb)

            @pl.when(g0 >= 2)
            def _():
                wait_out(out_vb, semob)

            compute(g0 + 1, bufb, out_vb, semob)
            return carry

        lax.fori_loop(0, _NCHUNK // 2, pair, 0)
        wait_out(out_va, semoa)
        wait_out(out_vb, semob)

    return k(value_flat, idx_flat, wts_flat)


def _prep_call(query, query_location, W_so, b_so, W_aw, b_aw):
    q2 = query.reshape(_BS * _NQ, _D)
    ql2 = query_location.reshape(_BS * _NQ, _L * 2)
    grid = (_BS * _NQ // _BQ,)
    full = lambda i: (0, 0)
    row = lambda i: (i, 0)
    return pl.pallas_call(
        _prep_body,
        grid=grid,
        in_specs=[
            pl.BlockSpec((_BQ, _D), row),
            pl.BlockSpec((_BQ, _L * 2), row),
            pl.BlockSpec((3 * _NCOMBO, _D), full),
            pl.BlockSpec((1, 3 * _NCOMBO), full),
        ],
        out_specs=[pl.BlockSpec((_BQ, _NCOMBO * 4), row)] * 2,
        out_shape=[
            jax.ShapeDtypeStruct((_BS * _NQ, _NCOMBO * 4), jnp.int32),
            jax.ShapeDtypeStruct((_BS * _NQ, _NCOMBO * 4), jnp.float32),
        ],
    )(q2, ql2,
      jnp.concatenate([W_so[0::2], W_so[1::2], W_aw], axis=0),
      jnp.concatenate([b_so[0::2], b_so[1::2], b_aw]).reshape(1, 3 * _NCOMBO))


def _proj_call(attn2, W_op, b_op):
    grid = (_BS * _NQ // _BQ,)
    return pl.pallas_call(
        _proj_body,
        grid=grid,
        in_specs=[
            pl.BlockSpec((_BQ, _H * _D), lambda i: (i, 0)),
            pl.BlockSpec((_D, _H * _D), lambda i: (0, 0)),
            pl.BlockSpec((1, _D), lambda i: (0, 0)),
        ],
        out_specs=pl.BlockSpec((_BQ, _D), lambda i: (i, 0)),
        out_shape=jax.ShapeDtypeStruct((_BS * _NQ, _D), jnp.float32),
    )(attn2, W_op, b_op.reshape(1, _D))


def kernel(query, value, query_location, spatial_shapes, level_start_index,
           W_so, b_so, W_aw, b_aw, W_op, b_op):
    idx, wt = _prep_call(query, query_location, W_so, b_so, W_aw, b_aw)
    value_flat = value.reshape(_BS * _NV, _D)
    attn2 = _sc_attend(value_flat, idx.reshape(-1), wt.reshape(-1))
    out = _proj_call(attn2, W_op, b_op)
    return out.reshape(_BS, _NQ, _D)
